# baseline (speedup 1.0000x reference)
import functools

import jax
import jax.numpy as jnp
from jax.experimental import pallas as pl
from jax.experimental.pallas import tpu as pltpu

LANE = 128
_VMEM_LIMIT = 44 * 1024 * 1024


def _round_up(x, m):
    return (x + m - 1) // m * m


def _pick_tile(dim_p, pref):
    t = max(LANE, min(pref, dim_p))
    t = (t // LANE) * LANE
    while dim_p % t:
        t -= LANE
    return t


def _gcn_body(alpha_ref, adj_ref, seq_ref, w_ref, bias_ref, o_ref, xw_ref):
    i = pl.program_id(1)

    @pl.when(i == 0)
    def _():
        xw_ref[...] = jnp.dot(
            seq_ref[...].astype(jnp.bfloat16), w_ref[...],
            preferred_element_type=jnp.float32).astype(jnp.bfloat16)

    @pl.when(i > 0)
    def _():
        h = jnp.dot(adj_ref[...].astype(jnp.bfloat16), xw_ref[...],
                    preferred_element_type=jnp.float32) + bias_ref[...]
        alpha = alpha_ref[0]
        o_ref[...] = jnp.where(h > 0.0, h, alpha * h)


@jax.jit
def kernel(seq, adj, w, bias, alpha):
    B, N, F_in = seq.shape
    F_h = w.shape[1]
    alpha1d = jnp.asarray(alpha, jnp.float32).reshape(1)

    Np = _round_up(N, LANE)
    Fi = _round_up(F_in, LANE)
    Fh = _round_up(F_h, LANE)
    seq_p = jnp.pad(seq.astype(jnp.float32), ((0, 0), (0, Np - N), (0, Fi - F_in)))
    adj_p = jnp.pad(adj.astype(jnp.float32), ((0, 0), (0, Np - N), (0, Np - N)))
    w_p = jnp.pad(w, ((0, Fi - F_in), (0, Fh - F_h))).astype(jnp.bfloat16)
    bias_p = jnp.pad(bias, (0, Fh - F_h)).reshape(1, Fh).astype(jnp.float32)

    tn = _pick_tile(Np, 1024)

    def _vmem_bytes(tn_):
        return (2 * tn_ * Np * 4
                + 2 * Np * Fi * 4
                + 2 * tn_ * Fh * 4
                + Np * Fh * 2
                + Np * Fh * 4
                + 2 * Fi * Fh * 2)

    while _vmem_bytes(tn) > _VMEM_LIMIT - 2 * 1024 * 1024 and tn > LANE:
        tn = _pick_tile(Np, tn - LANE)

    ni = Np // tn
    grid = (B, ni + 1)
    flops = 2 * B * (Np * Fi * Fh + Np * Np * Fh)
    bytes_accessed = (B * (Np * Np + Np * Fi + Np * Fh) * 4
                      + Fi * Fh * 2 + Fh * 4)

    def _adj_idx(b, i):
        return (b, jnp.maximum(i - 1, 0), 0)

    def _seq_idx(b, i):
        return (jnp.minimum(jnp.where(i >= ni, b + 1, b), B - 1), 0, 0)

    def _out_idx(b, i):
        return (b, jnp.maximum(i - 1, 0), 0)

    out = pl.pallas_call(
        _gcn_body,
        out_shape=jax.ShapeDtypeStruct((B, Np, Fh), jnp.float32),
        grid=grid,
        in_specs=[
            pl.BlockSpec(memory_space=pltpu.MemorySpace.SMEM),
            pl.BlockSpec((None, tn, Np), _adj_idx),
            pl.BlockSpec((None, Np, Fi), _seq_idx),
            pl.BlockSpec((Fi, Fh), lambda b, i: (0, 0)),
            pl.BlockSpec((1, Fh), lambda b, i: (0, 0)),
        ],
        out_specs=pl.BlockSpec((None, tn, Fh), _out_idx),
        scratch_shapes=[pltpu.VMEM((Np, Fh), jnp.bfloat16)],
        compiler_params=pltpu.CompilerParams(
            dimension_semantics=("parallel", "arbitrary"),
            vmem_limit_bytes=_VMEM_LIMIT),
        cost_estimate=pl.CostEstimate(flops=flops, transcendentals=0,
                                      bytes_accessed=bytes_accessed),
    )(alpha1d, adj_p, seq_p, w_p, bias_p)
    return out[:, :N, :F_h]


# speedup vs baseline: 1.0434x; 1.0434x over previous
import functools

import jax
import jax.numpy as jnp
from jax.experimental import pallas as pl
from jax.experimental.pallas import tpu as pltpu

LANE = 128
_VMEM_LIMIT = 44 * 1024 * 1024


def _round_up(x, m):
    return (x + m - 1) // m * m


def _pick_tile(dim_p, pref):
    t = max(LANE, min(pref, dim_p))
    t = (t // LANE) * LANE
    while dim_p % t:
        t -= LANE
    return t


def _make_body(ni, num_b):
    def _gcn_body(alpha_ref, adj_ref, seq_ref, w_ref, bias_ref, o_ref, xw_ref):
        b = pl.program_id(0)
        i = pl.program_id(1)
        slot = jax.lax.rem(b, 2)

        def _xw(dst_slot):
            xw_ref[dst_slot] = jnp.dot(
                seq_ref[...].astype(jnp.bfloat16), w_ref[...],
                preferred_element_type=jnp.float32).astype(jnp.bfloat16)

        @pl.when(jnp.logical_and(b == 0, i == 0))
        def _():
            _xw(0)

        @pl.when(i < ni)
        def _():
            h = jnp.dot(adj_ref[...].astype(jnp.bfloat16), xw_ref[slot],
                        preferred_element_type=jnp.float32) + bias_ref[...]
            alpha = alpha_ref[0]
            o_ref[...] = jnp.where(h > 0.0, h, alpha * h)

        @pl.when(jnp.logical_and(i == ni, b < num_b - 1))
        def _():
            _xw(1 - slot)

    return _gcn_body


@jax.jit
def kernel(seq, adj, w, bias, alpha):
    B, N, F_in = seq.shape
    F_h = w.shape[1]
    alpha1d = jnp.asarray(alpha, jnp.float32).reshape(1)

    Np = _round_up(N, LANE)
    Fi = _round_up(F_in, LANE)
    Fh = _round_up(F_h, LANE)
    seq_p = jnp.pad(seq.astype(jnp.float32), ((0, 0), (0, Np - N), (0, Fi - F_in)))
    adj_p = jnp.pad(adj.astype(jnp.float32), ((0, 0), (0, Np - N), (0, Np - N)))
    w_p = jnp.pad(w, ((0, Fi - F_in), (0, Fh - F_h))).astype(jnp.bfloat16)
    bias_p = jnp.pad(bias, (0, Fh - F_h)).reshape(1, Fh).astype(jnp.float32)

    tn = _pick_tile(Np, 1024)

    def _vmem_bytes(tn_):
        return (2 * tn_ * Np * 4
                + 2 * Np * Fi * 4
                + 2 * tn_ * Fh * 4
                + 2 * Np * Fh * 2
                + Np * Fh * 4
                + 2 * Fi * Fh * 2)

    while _vmem_bytes(tn) > _VMEM_LIMIT - 2 * 1024 * 1024 and tn > LANE:
        tn = _pick_tile(Np, tn - LANE)

    ni = Np // tn
    grid = (B, ni + 1)
    flops = 2 * B * (Np * Fi * Fh + Np * Np * Fh)
    bytes_accessed = (B * (Np * Np + Np * Fi + Np * Fh) * 4
                      + Fi * Fh * 2 + Fh * 4)

    def _adj_idx(b, i):
        return (b, jnp.minimum(i, ni - 1), 0)

    def _seq_idx(b, i):
        return (jnp.minimum(jnp.where(i >= ni, b + 1, b), B - 1), 0, 0)

    def _out_idx(b, i):
        return (b, jnp.minimum(i, ni - 1), 0)

    out = pl.pallas_call(
        _make_body(ni, B),
        out_shape=jax.ShapeDtypeStruct((B, Np, Fh), jnp.float32),
        grid=grid,
        in_specs=[
            pl.BlockSpec(memory_space=pltpu.MemorySpace.SMEM),
            pl.BlockSpec((None, tn, Np), _adj_idx),
            pl.BlockSpec((None, Np, Fi), _seq_idx),
            pl.BlockSpec((Fi, Fh), lambda b, i: (0, 0)),
            pl.BlockSpec((1, Fh), lambda b, i: (0, 0)),
        ],
        out_specs=pl.BlockSpec((None, tn, Fh), _out_idx),
        scratch_shapes=[pltpu.VMEM((2, Np, Fh), jnp.bfloat16)],
        compiler_params=pltpu.CompilerParams(
            dimension_semantics=("arbitrary", "arbitrary"),
            vmem_limit_bytes=_VMEM_LIMIT),
        cost_estimate=pl.CostEstimate(flops=flops, transcendentals=0,
                                      bytes_accessed=bytes_accessed),
    )(alpha1d, adj_p, seq_p, w_p, bias_p)
    return out[:, :N, :F_h]
